# baseline (speedup 1.0000x reference)
import functools
import jax
import jax.numpy as jnp
from jax import lax
from jax.experimental import pallas as pl
from jax.experimental.pallas import tpu as pltpu
from jax.experimental.pallas import tpu_sc as plsc

N = 10000
NP = 10240
E = 320000
D = 128
H = 128

NW = 32
EP = E // NW
C = 80
NCH = EP // C
TPN = NP // 16

@functools.cache
def _mesh():
    return plsc.VectorSubcoreMesh(core_axis_name="c", subcore_axis_name="s")


def _silu(v):
    return v * jax.nn.sigmoid(v)


def _layernorm(v, g, b, eps=1e-5):
    mu = jnp.mean(v, axis=-1, keepdims=True)
    var = jnp.mean((v - mu) ** 2, axis=-1, keepdims=True)
    return (v - mu) / jnp.sqrt(var + eps) * g + b


def _prep_body(h_ref, g1_ref, b1_ref, wa_ref, wb_ref, be1_ref,
               hn_ref, a_ref, b_ref):
    hn = _layernorm(h_ref[...], g1_ref[...], b1_ref[...])
    hn_ref[...] = hn
    a_ref[...] = (jnp.dot(hn, wa_ref[...], preferred_element_type=jnp.float32)
                  + be1_ref[...]).astype(jnp.bfloat16)
    b_ref[...] = jnp.dot(hn, wb_ref[...],
                         preferred_element_type=jnp.float32).astype(jnp.bfloat16)


def _tc_prep(h_p, g1, beta1, wa, wb, be1):
    BN = 1024
    grid = NP // BN
    return pl.pallas_call(
        _prep_body,
        grid=(grid,),
        in_specs=[
            pl.BlockSpec((BN, D), lambda i: (i, 0)),
            pl.BlockSpec((1, D), lambda i: (0, 0)),
            pl.BlockSpec((1, D), lambda i: (0, 0)),
            pl.BlockSpec((D, H), lambda i: (0, 0)),
            pl.BlockSpec((D, H), lambda i: (0, 0)),
            pl.BlockSpec((1, H), lambda i: (0, 0)),
        ],
        out_specs=[
            pl.BlockSpec((BN, D), lambda i: (i, 0)),
            pl.BlockSpec((BN, H), lambda i: (i, 0)),
            pl.BlockSpec((BN, H), lambda i: (i, 0)),
        ],
        out_shape=[
            jax.ShapeDtypeStruct((NP, D), jnp.float32),
            jax.ShapeDtypeStruct((NP, H), jnp.bfloat16),
            jax.ShapeDtypeStruct((NP, H), jnp.bfloat16),
        ],
    )(h_p, g1.reshape(1, D), beta1.reshape(1, D), wa, wb, be1.reshape(1, H))


def _rsqrt_sc(r2):
    y = plsc.bitcast(r2, jnp.int32)
    y = jnp.int32(0x5F3759DF) - lax.shift_right_logical(y, 1)
    g = plsc.bitcast(y, jnp.float32)
    g = g * (1.5 - 0.5 * r2 * g * g)
    g = g * (1.5 - 0.5 * r2 * g * g)
    g = g * (1.5 - 0.5 * r2 * g * g)
    return g


def _sc_gather_body(a_hbm, b_hbm, x0_hbm, x1_hbm, x2_hbm, row_hbm, col_hbm,
                    wd_hbm, pre_hbm, cnt_hbm,
                    rowv, colv, x0v, x1v, x2v, wdv, onesv,
                    abuf0, abuf1, bbuf0, bbuf1, obuf0, obuf1, dv0, dv1, zbuf,
                    cnt_sh, sa0, sa1, sb0, sb1, so0, so1, scnt):
    cid = lax.axis_index("c")
    sid = lax.axis_index("s")
    wid = cid * 16 + sid
    base = wid * EP

    pltpu.sync_copy(row_hbm.at[wid], rowv)
    pltpu.sync_copy(col_hbm.at[wid], colv)
    pltpu.sync_copy(x0_hbm, x0v)
    pltpu.sync_copy(x1_hbm, x1v)
    pltpu.sync_copy(x2_hbm, x2v)
    pltpu.sync_copy(wd_hbm, wdv)

    for i in range(TPN // 16):
        zbuf[pl.ds(i * 16, 16)] = jnp.zeros((16,), jnp.float32)
    for i in range(C // 16):
        onesv[pl.ds(i * 16, 16)] = jnp.ones((16,), jnp.float32)
    pltpu.sync_copy(zbuf, cnt_sh.at[pl.ds(sid * TPN, TPN)])
    plsc.subcore_barrier()

    def _dist(c, dv):
        rv = rowv.at[c]
        cv = colv.at[c]
        for g in range(C // 16):
            r16 = rv[pl.ds(g * 16, 16)]
            c16 = cv[pl.ds(g * 16, 16)]
            dx = plsc.load_gather(x0v, [r16]) - plsc.load_gather(x0v, [c16])
            dy = plsc.load_gather(x1v, [r16]) - plsc.load_gather(x1v, [c16])
            dz = plsc.load_gather(x2v, [r16]) - plsc.load_gather(x2v, [c16])
            r2 = dx * dx + dy * dy + dz * dz
            dv[pl.ds(g * 16, 16)] = r2 * _rsqrt_sc(jnp.maximum(r2, 1e-30))

    def _start(c, ab, bb, sa, sb):
        pltpu.async_copy(a_hbm.at[rowv.at[c]], ab, sa)
        pltpu.async_copy(b_hbm.at[colv.at[c]], bb, sb)

    def _wait(c, ab, bb, sa, sb):
        pltpu.make_async_copy(a_hbm.at[rowv.at[c]], ab, sa).wait()
        pltpu.make_async_copy(b_hbm.at[colv.at[c]], bb, sb).wait()

    def _compute(ab, bb, ob, dv):
        @pl.loop(0, C, unroll=4)
        def _edge(e):
            dsp = plsc.load_gather(dv, [jnp.full((16,), e, jnp.int32)])
            dspb = plsc.pack(dsp, dsp, format=plsc.PackFormat.INTERLEAVED)
            for f in range(H // 32):
                sl = pl.ds(f * 16, 16)
                av = plsc.bitcast(ab[e, sl], jnp.bfloat16)
                bv = plsc.bitcast(bb[e, sl], jnp.bfloat16)
                wv = wdv[pl.ds(f * 32, 32)]
                ob[e, sl] = plsc.bitcast(av + bv + dspb * wv, jnp.int32)

    def _store(c, ob, so):
        pltpu.async_copy(ob, pre_hbm.at[pl.ds(base + c * C, C)], so)

    def _store_wait(c, ob, so):
        pltpu.make_async_copy(ob, pre_hbm.at[pl.ds(base + c * C, C)], so).wait()

    _start(0, abuf0, bbuf0, sa0, sb0)

    @pl.loop(0, NCH // 2)
    def _pair(k):
        c0 = k * 2
        c1 = c0 + 1
        _start(c1, abuf1, bbuf1, sa1, sb1)
        pltpu.async_copy(onesv, cnt_sh.at[colv.at[c0]], scnt, add=True)
        _dist(c0, dv0)
        _wait(c0, abuf0, bbuf0, sa0, sb0)

        @pl.when(k > 0)
        def _():
            _store_wait(c0, obuf0, so0)

        _compute(abuf0, bbuf0, obuf0, dv0)
        _store(c0, obuf0, so0)

        _start(c0 + 2, abuf0, bbuf0, sa0, sb0)
        pltpu.async_copy(onesv, cnt_sh.at[colv.at[c1]], scnt, add=True)
        _dist(c1, dv1)
        _wait(c1, abuf1, bbuf1, sa1, sb1)

        @pl.when(k > 0)
        def _():
            _store_wait(c1, obuf1, so1)

        _compute(abuf1, bbuf1, obuf1, dv1)
        _store(c1, obuf1, so1)

    cl = NCH - 1
    pltpu.async_copy(onesv, cnt_sh.at[colv.at[cl]], scnt, add=True)
    _dist(cl, dv0)
    _wait(cl, abuf0, bbuf0, sa0, sb0)
    _store_wait(cl, obuf0, so0)
    _compute(abuf0, bbuf0, obuf0, dv0)
    _store(cl, obuf0, so0)
    _store_wait(cl, obuf0, so0)
    _store_wait(cl, obuf1, so1)

    @pl.loop(0, NCH)
    def _drain(c):
        pltpu.make_async_copy(onesv, cnt_sh.at[colv.at[c]], scnt).wait()

    plsc.subcore_barrier()
    pltpu.sync_copy(cnt_sh.at[pl.ds(sid * TPN, TPN)],
                    cnt_hbm.at[cid, pl.ds(sid * TPN, TPN)])


def _sc_gather(a, b, x0, x1, x2, row_r, col_r, wd):
    f = pl.kernel(
        _sc_gather_body,
        out_type=[
            jax.ShapeDtypeStruct((E, H // 2), jnp.int32),
            jax.ShapeDtypeStruct((2, NP), jnp.float32),
        ],
        mesh=_mesh(),
        scratch_types=[
            pltpu.VMEM((NCH, C), jnp.int32),
            pltpu.VMEM((NCH, C), jnp.int32),
            pltpu.VMEM((N,), jnp.float32),
            pltpu.VMEM((N,), jnp.float32),
            pltpu.VMEM((N,), jnp.float32),
            pltpu.VMEM((H,), jnp.bfloat16),
            pltpu.VMEM((C,), jnp.float32),
            pltpu.VMEM((C, H // 2), jnp.int32),
            pltpu.VMEM((C, H // 2), jnp.int32),
            pltpu.VMEM((C, H // 2), jnp.int32),
            pltpu.VMEM((C, H // 2), jnp.int32),
            pltpu.VMEM((C, H // 2), jnp.int32),
            pltpu.VMEM((C, H // 2), jnp.int32),
            pltpu.VMEM((C,), jnp.float32),
            pltpu.VMEM((C,), jnp.float32),
            pltpu.VMEM((TPN,), jnp.float32),
            pltpu.VMEM_SHARED((NP,), jnp.float32),
            pltpu.SemaphoreType.DMA,
            pltpu.SemaphoreType.DMA,
            pltpu.SemaphoreType.DMA,
            pltpu.SemaphoreType.DMA,
            pltpu.SemaphoreType.DMA,
            pltpu.SemaphoreType.DMA,
            pltpu.SemaphoreType.DMA,
        ],
        compiler_params=pltpu.CompilerParams(needs_layout_passes=False,
                                             use_tc_tiling_on_sc=False),
    )
    return f(a, b, x0, x1, x2, row_r, col_r, wd)


_BE = 2560


def _edge_body(pre_ref, w2_ref, b2_ref, m_ref):
    t = _silu(pre_ref[...].astype(jnp.float32))
    m_ref[...] = _silu(jnp.dot(t, w2_ref[...], preferred_element_type=jnp.float32)
                       + b2_ref[...])


def _tc_edge(pre, w2, b2):
    grid = E // _BE
    return pl.pallas_call(
        _edge_body,
        grid=(grid,),
        in_specs=[
            pl.BlockSpec((_BE, H), lambda i: (i, 0)),
            pl.BlockSpec((H, H), lambda i: (0, 0)),
            pl.BlockSpec((1, H), lambda i: (0, 0)),
        ],
        out_specs=pl.BlockSpec((_BE, H), lambda i: (i, 0)),
        out_shape=jax.ShapeDtypeStruct((E, H), jnp.float32),
    )(pre, w2, b2.reshape(1, H))


def _sc_scatter_body(m_hbm, col_hbm, sums_hbm,
                     colv, mbuf0, mbuf1, sums_sh, sm0, sm1):
    cid = lax.axis_index("c")
    sid = lax.axis_index("s")
    wid = cid * 16 + sid
    base = wid * EP

    pltpu.sync_copy(col_hbm.at[wid], colv)

    @pl.loop(0, C)
    def _z(i):
        for k in range(H // 16):
            mbuf0[i, pl.ds(k * 16, 16)] = jnp.zeros((16,), jnp.float32)

    for j in range(TPN // C):
        pltpu.sync_copy(mbuf0, sums_sh.at[pl.ds(sid * TPN + j * C, C)])
    plsc.subcore_barrier()

    def _load(c, mb, sm):
        pltpu.async_copy(m_hbm.at[pl.ds(base + c * C, C)], mb, sm)

    def _load_wait(c, mb, sm):
        pltpu.make_async_copy(m_hbm.at[pl.ds(base + c * C, C)], mb, sm).wait()

    _load(0, mbuf0, sm0)

    @pl.loop(0, NCH // 2)
    def _pair(k):
        c0 = k * 2
        _load(c0 + 1, mbuf1, sm1)
        _load_wait(c0, mbuf0, sm0)
        pltpu.sync_copy(mbuf0, sums_sh.at[colv.at[c0]], add=True)
        _load(c0 + 2, mbuf0, sm0)
        _load_wait(c0 + 1, mbuf1, sm1)
        pltpu.sync_copy(mbuf1, sums_sh.at[colv.at[c0 + 1]], add=True)

    cl = NCH - 1
    _load_wait(cl, mbuf0, sm0)
    pltpu.sync_copy(mbuf0, sums_sh.at[colv.at[cl]], add=True)

    plsc.subcore_barrier()
    pltpu.sync_copy(sums_sh.at[pl.ds(sid * TPN, TPN)],
                    sums_hbm.at[cid, pl.ds(sid * TPN, TPN)])


def _sc_scatter(m, col_r):
    f = pl.kernel(
        _sc_scatter_body,
        out_type=jax.ShapeDtypeStruct((2, NP, H), jnp.float32),
        mesh=_mesh(),
        scratch_types=[
            pltpu.VMEM((NCH, C), jnp.int32),
            pltpu.VMEM((C, H), jnp.float32),
            pltpu.VMEM((C, H), jnp.float32),
            pltpu.VMEM_SHARED((NP, H), jnp.float32),
            pltpu.SemaphoreType.DMA,
            pltpu.SemaphoreType.DMA,
        ],
        compiler_params=pltpu.CompilerParams(needs_layout_passes=False),
    )
    return f(m, col_r)


def _final_body(h_ref, hn_ref, s_ref, c_ref, wn1a_ref, wn1b_ref, bn1_ref,
                wn2_ref, bn2_ref, wm1_ref, bm1_ref, wm2_ref, bm2_ref,
                g2_ref, beta2_ref, out_ref):
    sums = s_ref[0] + s_ref[1]
    cnt = c_ref[0, 0] + c_ref[0, 1]
    m_aggr = sums / jnp.maximum(cnt, 1.0)[:, None]
    hn = hn_ref[...]
    pre_n = (jnp.dot(hn, wn1a_ref[...], preferred_element_type=jnp.float32)
             + jnp.dot(m_aggr, wn1b_ref[...], preferred_element_type=jnp.float32)
             + bn1_ref[...])
    h_delta = jnp.dot(_silu(pre_n), wn2_ref[...],
                      preferred_element_type=jnp.float32) + bn2_ref[...]
    h1 = h_ref[...] + hn + h_delta
    h2n = _layernorm(h1, g2_ref[...], beta2_ref[...])
    h_mlp = jnp.dot(_silu(jnp.dot(h2n, wm1_ref[...],
                                  preferred_element_type=jnp.float32)
                          + bm1_ref[...]),
                    wm2_ref[...], preferred_element_type=jnp.float32) + bm2_ref[...]
    out_ref[...] = h1 + h_mlp


def _tc_final(h_p, hn, sums2, cnt2, wn1a, wn1b, b_n1, w_n2, b_n2,
              w_m1, b_m1, w_m2, b_m2, g2, beta2):
    BN = 1024
    grid = NP // BN
    cnt3 = cnt2.reshape(1, 2, NP)
    return pl.pallas_call(
        _final_body,
        grid=(grid,),
        in_specs=[
            pl.BlockSpec((BN, D), lambda i: (i, 0)),
            pl.BlockSpec((BN, D), lambda i: (i, 0)),
            pl.BlockSpec((2, BN, H), lambda i: (0, i, 0)),
            pl.BlockSpec((1, 2, BN), lambda i: (0, 0, i)),
            pl.BlockSpec((D, H), lambda i: (0, 0)),
            pl.BlockSpec((H, H), lambda i: (0, 0)),
            pl.BlockSpec((1, H), lambda i: (0, 0)),
            pl.BlockSpec((H, D), lambda i: (0, 0)),
            pl.BlockSpec((1, D), lambda i: (0, 0)),
            pl.BlockSpec((D, H), lambda i: (0, 0)),
            pl.BlockSpec((1, H), lambda i: (0, 0)),
            pl.BlockSpec((H, D), lambda i: (0, 0)),
            pl.BlockSpec((1, D), lambda i: (0, 0)),
            pl.BlockSpec((1, D), lambda i: (0, 0)),
            pl.BlockSpec((1, D), lambda i: (0, 0)),
        ],
        out_specs=pl.BlockSpec((BN, D), lambda i: (i, 0)),
        out_shape=jax.ShapeDtypeStruct((NP, D), jnp.float32),
    )(h_p, hn, sums2, cnt3, wn1a, wn1b, b_n1.reshape(1, H), w_n2,
      b_n2.reshape(1, D), w_m1, b_m1.reshape(1, H), w_m2,
      b_m2.reshape(1, D), g2.reshape(1, D), beta2.reshape(1, D))


def kernel(x, h, edge_index, W_e1, b_e1, W_e2, b_e2, W_n1, b_n1, W_n2, b_n2,
           W_m1, b_m1, W_m2, b_m2, g1, beta1, g2, beta2):
    row = edge_index[0].astype(jnp.int32)
    col = edge_index[1].astype(jnp.int32)
    row_r = row.reshape(NW, NCH, C)
    col_r = col.reshape(NW, NCH, C)

    h_p = jnp.zeros((NP, D), jnp.float32).at[:N].set(h)
    x0, x1, x2 = x[:, 0], x[:, 1], x[:, 2]

    wa = W_e1[:D]
    wb = W_e1[D:2 * D]
    wd = W_e1[2 * D].astype(jnp.bfloat16)

    hn, a, b = _tc_prep(h_p, g1, beta1, wa, wb, b_e1)
    ai = lax.bitcast_convert_type(a.reshape(NP, H // 2, 2), jnp.int32)
    bi = lax.bitcast_convert_type(b.reshape(NP, H // 2, 2), jnp.int32)
    pre_i, cnt2 = _sc_gather(ai, bi, x0, x1, x2, row_r, col_r, wd)
    pre = lax.bitcast_convert_type(pre_i, jnp.bfloat16).reshape(E, H)
    m = _tc_edge(pre, W_e2, b_e2)
    sums2 = _sc_scatter(m, col_r)
    out_p = _tc_final(h_p, hn, sums2, cnt2, W_n1[:D], W_n1[D:], b_n1,
                      W_n2, b_n2, W_m1, b_m1, W_m2, b_m2, g2, beta2)
    return out_p[:N]


# speedup vs baseline: 1.6325x; 1.6325x over previous
import functools
import jax
import jax.numpy as jnp
from jax import lax
from jax.experimental import pallas as pl
from jax.experimental.pallas import tpu as pltpu
from jax.experimental.pallas import tpu_sc as plsc

N = 10000
NP = 10240
E = 320000
D = 128
H = 128

NW = 32
EP = E // NW
C = 80
NCH = EP // C
TPN = NP // 16

@functools.cache
def _mesh():
    return plsc.VectorSubcoreMesh(core_axis_name="c", subcore_axis_name="s")


def _silu(v):
    return v * jax.nn.sigmoid(v)


def _layernorm(v, g, b, eps=1e-5):
    mu = jnp.mean(v, axis=-1, keepdims=True)
    var = jnp.mean((v - mu) ** 2, axis=-1, keepdims=True)
    return (v - mu) / jnp.sqrt(var + eps) * g + b


def _prep_body(h_ref, g1_ref, b1_ref, wa_ref, wb_ref, be1_ref,
               hn_ref, a_ref, b_ref):
    hn = _layernorm(h_ref[...], g1_ref[...], b1_ref[...])
    hn_ref[...] = hn
    a_ref[...] = jnp.dot(hn, wa_ref[...], preferred_element_type=jnp.float32) + be1_ref[...]
    b_ref[...] = jnp.dot(hn, wb_ref[...], preferred_element_type=jnp.float32)


def _tc_prep(h_p, g1, beta1, wa, wb, be1):
    BN = 1024
    grid = NP // BN
    return pl.pallas_call(
        _prep_body,
        grid=(grid,),
        in_specs=[
            pl.BlockSpec((BN, D), lambda i: (i, 0)),
            pl.BlockSpec((1, D), lambda i: (0, 0)),
            pl.BlockSpec((1, D), lambda i: (0, 0)),
            pl.BlockSpec((D, H), lambda i: (0, 0)),
            pl.BlockSpec((D, H), lambda i: (0, 0)),
            pl.BlockSpec((1, H), lambda i: (0, 0)),
        ],
        out_specs=[
            pl.BlockSpec((BN, D), lambda i: (i, 0)),
            pl.BlockSpec((BN, H), lambda i: (i, 0)),
            pl.BlockSpec((BN, H), lambda i: (i, 0)),
        ],
        out_shape=[
            jax.ShapeDtypeStruct((NP, D), jnp.float32),
            jax.ShapeDtypeStruct((NP, H), jnp.float32),
            jax.ShapeDtypeStruct((NP, H), jnp.float32),
        ],
    )(h_p, g1.reshape(1, D), beta1.reshape(1, D), wa, wb, be1.reshape(1, H))


def _rsqrt_sc(r2):
    y = plsc.bitcast(r2, jnp.int32)
    y = jnp.int32(0x5F3759DF) - lax.shift_right_logical(y, 1)
    g = plsc.bitcast(y, jnp.float32)
    g = g * (1.5 - 0.5 * r2 * g * g)
    g = g * (1.5 - 0.5 * r2 * g * g)
    g = g * (1.5 - 0.5 * r2 * g * g)
    return g


def _sc_gather_body(a_hbm, b_hbm, x0_hbm, x1_hbm, x2_hbm, row_hbm, col_hbm,
                    wd_hbm, pre_hbm, cnt_hbm,
                    rowv, colv, x0v, x1v, x2v, wdv, onesv,
                    abuf0, abuf1, bbuf0, bbuf1, obuf0, obuf1, dv0, dv1, zbuf,
                    cnt_sh, sa0, sa1, sb0, sb1, so0, so1, scnt):
    cid = lax.axis_index("c")
    sid = lax.axis_index("s")
    wid = cid * 16 + sid
    base = wid * EP

    pltpu.sync_copy(row_hbm.at[wid], rowv)
    pltpu.sync_copy(col_hbm.at[wid], colv)
    pltpu.sync_copy(x0_hbm, x0v)
    pltpu.sync_copy(x1_hbm, x1v)
    pltpu.sync_copy(x2_hbm, x2v)
    pltpu.sync_copy(wd_hbm, wdv)

    for i in range(TPN // 16):
        zbuf[pl.ds(i * 16, 16)] = jnp.zeros((16,), jnp.float32)
    for i in range(C // 16):
        onesv[pl.ds(i * 16, 16)] = jnp.ones((16,), jnp.float32)
    pltpu.sync_copy(zbuf, cnt_sh.at[pl.ds(sid * TPN, TPN)])
    plsc.subcore_barrier()

    def _dist(c, dv):
        rv = rowv.at[c]
        cv = colv.at[c]
        for g in range(C // 16):
            r16 = rv[pl.ds(g * 16, 16)]
            c16 = cv[pl.ds(g * 16, 16)]
            dx = plsc.load_gather(x0v, [r16]) - plsc.load_gather(x0v, [c16])
            dy = plsc.load_gather(x1v, [r16]) - plsc.load_gather(x1v, [c16])
            dz = plsc.load_gather(x2v, [r16]) - plsc.load_gather(x2v, [c16])
            r2 = dx * dx + dy * dy + dz * dz
            dv[pl.ds(g * 16, 16)] = r2 * _rsqrt_sc(jnp.maximum(r2, 1e-30))

    def _start(c, ab, bb, sa, sb):
        pltpu.async_copy(a_hbm.at[rowv.at[c]], ab, sa)
        pltpu.async_copy(b_hbm.at[colv.at[c]], bb, sb)

    def _wait(c, ab, bb, sa, sb):
        pltpu.make_async_copy(a_hbm.at[rowv.at[c]], ab, sa).wait()
        pltpu.make_async_copy(b_hbm.at[colv.at[c]], bb, sb).wait()

    def _compute(ab, bb, ob, dv):
        @pl.loop(0, C, unroll=4)
        def _edge(e):
            dsp = plsc.load_gather(dv, [jnp.full((16,), e, jnp.int32)])
            for f in range(H // 16):
                sl = pl.ds(f * 16, 16)
                ob[e, sl] = ab[e, sl] + bb[e, sl] + dsp * wdv[sl]

    def _store(c, ob, so):
        pltpu.async_copy(ob, pre_hbm.at[pl.ds(base + c * C, C)], so)

    def _store_wait(c, ob, so):
        pltpu.make_async_copy(ob, pre_hbm.at[pl.ds(base + c * C, C)], so).wait()

    _start(0, abuf0, bbuf0, sa0, sb0)

    @pl.loop(0, NCH // 2)
    def _pair(k):
        c0 = k * 2
        c1 = c0 + 1
        _start(c1, abuf1, bbuf1, sa1, sb1)
        pltpu.async_copy(onesv, cnt_sh.at[colv.at[c0]], scnt, add=True)
        _dist(c0, dv0)
        _wait(c0, abuf0, bbuf0, sa0, sb0)

        @pl.when(k > 0)
        def _():
            _store_wait(c0, obuf0, so0)

        _compute(abuf0, bbuf0, obuf0, dv0)
        _store(c0, obuf0, so0)

        _start(c0 + 2, abuf0, bbuf0, sa0, sb0)
        pltpu.async_copy(onesv, cnt_sh.at[colv.at[c1]], scnt, add=True)
        _dist(c1, dv1)
        _wait(c1, abuf1, bbuf1, sa1, sb1)

        @pl.when(k > 0)
        def _():
            _store_wait(c1, obuf1, so1)

        _compute(abuf1, bbuf1, obuf1, dv1)
        _store(c1, obuf1, so1)

    cl = NCH - 1
    pltpu.async_copy(onesv, cnt_sh.at[colv.at[cl]], scnt, add=True)
    _dist(cl, dv0)
    _wait(cl, abuf0, bbuf0, sa0, sb0)
    _store_wait(cl, obuf0, so0)
    _compute(abuf0, bbuf0, obuf0, dv0)
    _store(cl, obuf0, so0)
    _store_wait(cl, obuf0, so0)
    _store_wait(cl, obuf1, so1)

    @pl.loop(0, NCH)
    def _drain(c):
        pltpu.make_async_copy(onesv, cnt_sh.at[colv.at[c]], scnt).wait()

    plsc.subcore_barrier()
    pltpu.sync_copy(cnt_sh.at[pl.ds(sid * TPN, TPN)],
                    cnt_hbm.at[cid, pl.ds(sid * TPN, TPN)])


def _sc_gather(a, b, x0, x1, x2, row_r, col_r, wd):
    f = pl.kernel(
        _sc_gather_body,
        out_type=[
            jax.ShapeDtypeStruct((E, H), jnp.float32),
            jax.ShapeDtypeStruct((2, NP), jnp.float32),
        ],
        mesh=_mesh(),
        scratch_types=[
            pltpu.VMEM((NCH, C), jnp.int32),
            pltpu.VMEM((NCH, C), jnp.int32),
            pltpu.VMEM((N,), jnp.float32),
            pltpu.VMEM((N,), jnp.float32),
            pltpu.VMEM((N,), jnp.float32),
            pltpu.VMEM((H,), jnp.float32),
            pltpu.VMEM((C,), jnp.float32),
            pltpu.VMEM((C, H), jnp.float32),
            pltpu.VMEM((C, H), jnp.float32),
            pltpu.VMEM((C, H), jnp.float32),
            pltpu.VMEM((C, H), jnp.float32),
            pltpu.VMEM((C, H), jnp.float32),
            pltpu.VMEM((C, H), jnp.float32),
            pltpu.VMEM((C,), jnp.float32),
            pltpu.VMEM((C,), jnp.float32),
            pltpu.VMEM((TPN,), jnp.float32),
            pltpu.VMEM_SHARED((NP,), jnp.float32),
            pltpu.SemaphoreType.DMA,
            pltpu.SemaphoreType.DMA,
            pltpu.SemaphoreType.DMA,
            pltpu.SemaphoreType.DMA,
            pltpu.SemaphoreType.DMA,
            pltpu.SemaphoreType.DMA,
            pltpu.SemaphoreType.DMA,
        ],
        compiler_params=pltpu.CompilerParams(needs_layout_passes=False),
    )
    return f(a, b, x0, x1, x2, row_r, col_r, wd)


_BE = 2560


def _edge_body(pre_ref, w2_ref, b2_ref, m_ref):
    t = _silu(pre_ref[...])
    m_ref[...] = _silu(jnp.dot(t, w2_ref[...], preferred_element_type=jnp.float32)
                       + b2_ref[...])


def _tc_edge(pre, w2, b2):
    grid = E // _BE
    return pl.pallas_call(
        _edge_body,
        grid=(grid,),
        in_specs=[
            pl.BlockSpec((_BE, H), lambda i: (i, 0)),
            pl.BlockSpec((H, H), lambda i: (0, 0)),
            pl.BlockSpec((1, H), lambda i: (0, 0)),
        ],
        out_specs=pl.BlockSpec((_BE, H), lambda i: (i, 0)),
        out_shape=jax.ShapeDtypeStruct((E, H), jnp.float32),
    )(pre, w2, b2.reshape(1, H))


def _sc_scatter_body(m_hbm, col_hbm, sums_hbm,
                     colv, mbuf0, mbuf1, sums_sh, sm0, sm1):
    cid = lax.axis_index("c")
    sid = lax.axis_index("s")
    wid = cid * 16 + sid
    base = wid * EP

    pltpu.sync_copy(col_hbm.at[wid], colv)

    @pl.loop(0, C)
    def _z(i):
        for k in range(H // 16):
            mbuf0[i, pl.ds(k * 16, 16)] = jnp.zeros((16,), jnp.float32)

    for j in range(TPN // C):
        pltpu.sync_copy(mbuf0, sums_sh.at[pl.ds(sid * TPN + j * C, C)])
    plsc.subcore_barrier()

    def _load(c, mb, sm):
        pltpu.async_copy(m_hbm.at[pl.ds(base + c * C, C)], mb, sm)

    def _load_wait(c, mb, sm):
        pltpu.make_async_copy(m_hbm.at[pl.ds(base + c * C, C)], mb, sm).wait()

    _load(0, mbuf0, sm0)

    @pl.loop(0, NCH // 2)
    def _pair(k):
        c0 = k * 2
        _load(c0 + 1, mbuf1, sm1)
        _load_wait(c0, mbuf0, sm0)
        pltpu.sync_copy(mbuf0, sums_sh.at[colv.at[c0]], add=True)
        _load(c0 + 2, mbuf0, sm0)
        _load_wait(c0 + 1, mbuf1, sm1)
        pltpu.sync_copy(mbuf1, sums_sh.at[colv.at[c0 + 1]], add=True)

    cl = NCH - 1
    _load_wait(cl, mbuf0, sm0)
    pltpu.sync_copy(mbuf0, sums_sh.at[colv.at[cl]], add=True)

    plsc.subcore_barrier()
    pltpu.sync_copy(sums_sh.at[pl.ds(sid * TPN, TPN)],
                    sums_hbm.at[cid, pl.ds(sid * TPN, TPN)])


def _sc_scatter(m, col_r):
    f = pl.kernel(
        _sc_scatter_body,
        out_type=jax.ShapeDtypeStruct((2, NP, H), jnp.float32),
        mesh=_mesh(),
        scratch_types=[
            pltpu.VMEM((NCH, C), jnp.int32),
            pltpu.VMEM((C, H), jnp.float32),
            pltpu.VMEM((C, H), jnp.float32),
            pltpu.VMEM_SHARED((NP, H), jnp.float32),
            pltpu.SemaphoreType.DMA,
            pltpu.SemaphoreType.DMA,
        ],
        compiler_params=pltpu.CompilerParams(needs_layout_passes=False),
    )
    return f(m, col_r)


def _final_body(h_ref, hn_ref, s_ref, c_ref, wn1a_ref, wn1b_ref, bn1_ref,
                wn2_ref, bn2_ref, wm1_ref, bm1_ref, wm2_ref, bm2_ref,
                g2_ref, beta2_ref, out_ref):
    sums = s_ref[0] + s_ref[1]
    cnt = c_ref[0, 0] + c_ref[0, 1]
    m_aggr = sums / jnp.maximum(cnt, 1.0)[:, None]
    hn = hn_ref[...]
    pre_n = (jnp.dot(hn, wn1a_ref[...], preferred_element_type=jnp.float32)
             + jnp.dot(m_aggr, wn1b_ref[...], preferred_element_type=jnp.float32)
             + bn1_ref[...])
    h_delta = jnp.dot(_silu(pre_n), wn2_ref[...],
                      preferred_element_type=jnp.float32) + bn2_ref[...]
    h1 = h_ref[...] + hn + h_delta
    h2n = _layernorm(h1, g2_ref[...], beta2_ref[...])
    h_mlp = jnp.dot(_silu(jnp.dot(h2n, wm1_ref[...],
                                  preferred_element_type=jnp.float32)
                          + bm1_ref[...]),
                    wm2_ref[...], preferred_element_type=jnp.float32) + bm2_ref[...]
    out_ref[...] = h1 + h_mlp


def _tc_final(h_p, hn, sums2, cnt2, wn1a, wn1b, b_n1, w_n2, b_n2,
              w_m1, b_m1, w_m2, b_m2, g2, beta2):
    BN = 1024
    grid = NP // BN
    cnt3 = cnt2.reshape(1, 2, NP)
    return pl.pallas_call(
        _final_body,
        grid=(grid,),
        in_specs=[
            pl.BlockSpec((BN, D), lambda i: (i, 0)),
            pl.BlockSpec((BN, D), lambda i: (i, 0)),
            pl.BlockSpec((2, BN, H), lambda i: (0, i, 0)),
            pl.BlockSpec((1, 2, BN), lambda i: (0, 0, i)),
            pl.BlockSpec((D, H), lambda i: (0, 0)),
            pl.BlockSpec((H, H), lambda i: (0, 0)),
            pl.BlockSpec((1, H), lambda i: (0, 0)),
            pl.BlockSpec((H, D), lambda i: (0, 0)),
            pl.BlockSpec((1, D), lambda i: (0, 0)),
            pl.BlockSpec((D, H), lambda i: (0, 0)),
            pl.BlockSpec((1, H), lambda i: (0, 0)),
            pl.BlockSpec((H, D), lambda i: (0, 0)),
            pl.BlockSpec((1, D), lambda i: (0, 0)),
            pl.BlockSpec((1, D), lambda i: (0, 0)),
            pl.BlockSpec((1, D), lambda i: (0, 0)),
        ],
        out_specs=pl.BlockSpec((BN, D), lambda i: (i, 0)),
        out_shape=jax.ShapeDtypeStruct((NP, D), jnp.float32),
    )(h_p, hn, sums2, cnt3, wn1a, wn1b, b_n1.reshape(1, H), w_n2,
      b_n2.reshape(1, D), w_m1, b_m1.reshape(1, H), w_m2,
      b_m2.reshape(1, D), g2.reshape(1, D), beta2.reshape(1, D))


def kernel(x, h, edge_index, W_e1, b_e1, W_e2, b_e2, W_n1, b_n1, W_n2, b_n2,
           W_m1, b_m1, W_m2, b_m2, g1, beta1, g2, beta2):
    row = edge_index[0].astype(jnp.int32)
    col = edge_index[1].astype(jnp.int32)
    row_r = row.reshape(NW, NCH, C)
    col_r = col.reshape(NW, NCH, C)

    h_p = jnp.zeros((NP, D), jnp.float32).at[:N].set(h)
    x0, x1, x2 = x[:, 0], x[:, 1], x[:, 2]

    wa = W_e1[:D]
    wb = W_e1[D:2 * D]
    wd = W_e1[2 * D]

    hn, a, b = _tc_prep(h_p, g1, beta1, wa, wb, b_e1)
    pre, cnt2 = _sc_gather(a, b, x0, x1, x2, row_r, col_r, wd)
    m = _tc_edge(pre, W_e2, b_e2)
    sums2 = _sc_scatter(m, col_r)
    out_p = _tc_final(h_p, hn, sums2, cnt2, W_n1[:D], W_n1[D:], b_n1,
                      W_n2, b_n2, W_m1, b_m1, W_m2, b_m2, g2, beta2)
    return out_p[:N]


# speedup vs baseline: 1.9845x; 1.2156x over previous
import functools
import jax
import jax.numpy as jnp
import numpy as np
from jax import lax
from jax.experimental import pallas as pl
from jax.experimental.pallas import tpu as pltpu
from jax.experimental.pallas import tpu_sc as plsc

N = 10000
NP = 10240
E = 320000
D = 128
H = 128

NW = 32
EP = E // NW
C = 80
NCH = EP // C
TPN = NP // 16

@functools.cache
def _mesh():
    return plsc.VectorSubcoreMesh(core_axis_name="c", subcore_axis_name="s")


def _silu(v):
    return v * jax.nn.sigmoid(v)


def _layernorm(v, g, b, eps=1e-5):
    mu = jnp.mean(v, axis=-1, keepdims=True)
    var = jnp.mean((v - mu) ** 2, axis=-1, keepdims=True)
    return (v - mu) / jnp.sqrt(var + eps) * g + b


def _prep_body(h_ref, g1_ref, b1_ref, wa_ref, wb_ref, be1_ref,
               hn_ref, a_ref, b_ref):
    hn = _layernorm(h_ref[...], g1_ref[...], b1_ref[...])
    hn_ref[...] = hn
    a_ref[...] = (jnp.dot(hn, wa_ref[...], preferred_element_type=jnp.float32)
                  + be1_ref[...]).astype(jnp.bfloat16)
    b_ref[...] = jnp.dot(hn, wb_ref[...],
                         preferred_element_type=jnp.float32).astype(jnp.bfloat16)


def _tc_prep(h_p, g1, beta1, wa, wb, be1):
    BN = 1024
    grid = NP // BN
    return pl.pallas_call(
        _prep_body,
        grid=(grid,),
        in_specs=[
            pl.BlockSpec((BN, D), lambda i: (i, 0)),
            pl.BlockSpec((1, D), lambda i: (0, 0)),
            pl.BlockSpec((1, D), lambda i: (0, 0)),
            pl.BlockSpec((D, H), lambda i: (0, 0)),
            pl.BlockSpec((D, H), lambda i: (0, 0)),
            pl.BlockSpec((1, H), lambda i: (0, 0)),
        ],
        out_specs=[
            pl.BlockSpec((BN, D), lambda i: (i, 0)),
            pl.BlockSpec((BN, H), lambda i: (i, 0)),
            pl.BlockSpec((BN, H), lambda i: (i, 0)),
        ],
        out_shape=[
            jax.ShapeDtypeStruct((NP, D), jnp.float32),
            jax.ShapeDtypeStruct((NP, H), jnp.bfloat16),
            jax.ShapeDtypeStruct((NP, H), jnp.bfloat16),
        ],
    )(h_p, g1.reshape(1, D), beta1.reshape(1, D), wa, wb, be1.reshape(1, H))


def _rsqrt_sc(r2):
    y = plsc.bitcast(r2, jnp.int32)
    y = jnp.int32(0x5F3759DF) - lax.shift_right_logical(y, 1)
    g = plsc.bitcast(y, jnp.float32)
    g = g * (1.5 - 0.5 * r2 * g * g)
    g = g * (1.5 - 0.5 * r2 * g * g)
    g = g * (1.5 - 0.5 * r2 * g * g)
    return g


def _sc_gather_body(abp_hbm, x0_hbm, x1_hbm, x2_hbm, row_hbm, col_hbm,
                    wd_hbm, pre_hbm, cnt_hbm,
                    rowv, colv, x0v, x1v, x2v, wdv, onesv,
                    abuf0, abuf1, bbuf0, bbuf1, obuf0, obuf1, dv0, dv1, zbuf,
                    cnt_sh, sa0, sa1, sb0, sb1, so0, so1, scnt):
    cid = lax.axis_index("c")
    sid = lax.axis_index("s")
    wid = cid * 16 + sid
    base = wid * EP

    pltpu.sync_copy(row_hbm.at[wid], rowv)
    pltpu.sync_copy(col_hbm.at[wid], colv)
    pltpu.sync_copy(x0_hbm, x0v)
    pltpu.sync_copy(x1_hbm, x1v)
    pltpu.sync_copy(x2_hbm, x2v)
    pltpu.sync_copy(wd_hbm, wdv)

    for i in range(TPN // 16):
        zbuf[pl.ds(i * 16, 16)] = jnp.zeros((16,), jnp.float32)
    for i in range(C // 16):
        onesv[pl.ds(i * 16, 16)] = jnp.ones((16,), jnp.float32)
    pltpu.sync_copy(zbuf, cnt_sh.at[pl.ds(sid * TPN, TPN)])
    plsc.subcore_barrier()

    def _dist(c, dv):
        rv = rowv.at[c]
        cv = colv.at[c]
        for g in range(C // 16):
            r16 = rv[pl.ds(g * 16, 16)]
            c16 = cv[pl.ds(g * 16, 16)]
            dx = plsc.load_gather(x0v, [r16]) - plsc.load_gather(x0v, [c16])
            dy = plsc.load_gather(x1v, [r16]) - plsc.load_gather(x1v, [c16])
            dz = plsc.load_gather(x2v, [r16]) - plsc.load_gather(x2v, [c16])
            r2 = dx * dx + dy * dy + dz * dz
            dv[pl.ds(g * 16, 16)] = r2 * _rsqrt_sc(jnp.maximum(r2, 1e-30))

    def _start(c, ab, bb, sa, sb):
        pltpu.async_copy(abp_hbm.at[rowv.at[c]], ab, sa)
        pltpu.async_copy(abp_hbm.at[colv.at[c]], bb, sb)

    def _wait(c, ab, bb, sa, sb):
        pltpu.make_async_copy(abp_hbm.at[rowv.at[c]], ab, sa).wait()
        pltpu.make_async_copy(abp_hbm.at[colv.at[c]], bb, sb).wait()

    def _compute(ab, bb, ob, dv):
        @pl.loop(0, C, unroll=4)
        def _edge(e):
            dsp = plsc.load_gather(dv, [jnp.full((16,), e, jnp.int32)])
            dspb = plsc.pack(dsp, dsp, format=plsc.PackFormat.INTERLEAVED)
            for f in range(H // 32):
                av = plsc.bitcast(ab[e, pl.ds(f * 16, 16)], jnp.bfloat16)
                bv = plsc.bitcast(bb[e, pl.ds(64 + f * 16, 16)], jnp.bfloat16)
                wv = plsc.bitcast(wdv[pl.ds(f * 16, 16)], jnp.bfloat16)
                res = av + bv + dspb * wv
                lo, hi = plsc.unpack(res, format=plsc.PackFormat.INTERLEAVED)
                ob[e, pl.ds(f * 32, 16)] = lo
                ob[e, pl.ds(f * 32 + 16, 16)] = hi

    def _store(c, ob, so):
        pltpu.async_copy(ob, pre_hbm.at[pl.ds(base + c * C, C)], so)

    def _store_wait(c, ob, so):
        pltpu.make_async_copy(ob, pre_hbm.at[pl.ds(base + c * C, C)], so).wait()

    _start(0, abuf0, bbuf0, sa0, sb0)

    @pl.loop(0, NCH // 2)
    def _pair(k):
        c0 = k * 2
        c1 = c0 + 1
        _start(c1, abuf1, bbuf1, sa1, sb1)
        pltpu.async_copy(onesv, cnt_sh.at[colv.at[c0]], scnt, add=True)
        _dist(c0, dv0)
        _wait(c0, abuf0, bbuf0, sa0, sb0)

        @pl.when(k > 0)
        def _():
            _store_wait(c0, obuf0, so0)

        _compute(abuf0, bbuf0, obuf0, dv0)
        _store(c0, obuf0, so0)

        _start(c0 + 2, abuf0, bbuf0, sa0, sb0)
        pltpu.async_copy(onesv, cnt_sh.at[colv.at[c1]], scnt, add=True)
        _dist(c1, dv1)
        _wait(c1, abuf1, bbuf1, sa1, sb1)

        @pl.when(k > 0)
        def _():
            _store_wait(c1, obuf1, so1)

        _compute(abuf1, bbuf1, obuf1, dv1)
        _store(c1, obuf1, so1)

    cl = NCH - 1
    pltpu.async_copy(onesv, cnt_sh.at[colv.at[cl]], scnt, add=True)
    _dist(cl, dv0)
    _wait(cl, abuf0, bbuf0, sa0, sb0)
    _store_wait(cl, obuf0, so0)
    _compute(abuf0, bbuf0, obuf0, dv0)
    _store(cl, obuf0, so0)
    _store_wait(cl, obuf0, so0)
    _store_wait(cl, obuf1, so1)

    @pl.loop(0, NCH)
    def _drain(c):
        pltpu.make_async_copy(onesv, cnt_sh.at[colv.at[c]], scnt).wait()

    plsc.subcore_barrier()
    pltpu.sync_copy(cnt_sh.at[pl.ds(sid * TPN, TPN)],
                    cnt_hbm.at[cid, pl.ds(sid * TPN, TPN)])


def _sc_gather(abp, x0, x1, x2, row_r, col_r, wdi):
    f = pl.kernel(
        _sc_gather_body,
        out_type=[
            jax.ShapeDtypeStruct((E, H), jnp.float32),
            jax.ShapeDtypeStruct((2, NP), jnp.float32),
        ],
        mesh=_mesh(),
        scratch_types=[
            pltpu.VMEM((NCH, C), jnp.int32),
            pltpu.VMEM((NCH, C), jnp.int32),
            pltpu.VMEM((N,), jnp.float32),
            pltpu.VMEM((N,), jnp.float32),
            pltpu.VMEM((N,), jnp.float32),
            pltpu.VMEM((H // 2,), jnp.int32),
            pltpu.VMEM((C,), jnp.float32),
            pltpu.VMEM((C, H), jnp.int32),
            pltpu.VMEM((C, H), jnp.int32),
            pltpu.VMEM((C, H), jnp.int32),
            pltpu.VMEM((C, H), jnp.int32),
            pltpu.VMEM((C, H), jnp.float32),
            pltpu.VMEM((C, H), jnp.float32),
            pltpu.VMEM((C,), jnp.float32),
            pltpu.VMEM((C,), jnp.float32),
            pltpu.VMEM((TPN,), jnp.float32),
            pltpu.VMEM_SHARED((NP,), jnp.float32),
            pltpu.SemaphoreType.DMA,
            pltpu.SemaphoreType.DMA,
            pltpu.SemaphoreType.DMA,
            pltpu.SemaphoreType.DMA,
            pltpu.SemaphoreType.DMA,
            pltpu.SemaphoreType.DMA,
            pltpu.SemaphoreType.DMA,
        ],
        compiler_params=pltpu.CompilerParams(needs_layout_passes=False,
                                             disable_bounds_checks=True),
    )
    return f(abp, x0, x1, x2, row_r, col_r, wdi)


_BE = 2560


def _edge_body(pre_ref, w2_ref, b2_ref, m_ref):
    t = _silu(pre_ref[...])
    m_ref[...] = _silu(jnp.dot(t, w2_ref[...], preferred_element_type=jnp.float32)
                       + b2_ref[...])


def _tc_edge(pre, w2, b2):
    grid = E // _BE
    return pl.pallas_call(
        _edge_body,
        grid=(grid,),
        in_specs=[
            pl.BlockSpec((_BE, H), lambda i: (i, 0)),
            pl.BlockSpec((H, H), lambda i: (0, 0)),
            pl.BlockSpec((1, H), lambda i: (0, 0)),
        ],
        out_specs=pl.BlockSpec((_BE, H), lambda i: (i, 0)),
        out_shape=jax.ShapeDtypeStruct((E, H), jnp.float32),
    )(pre, w2, b2.reshape(1, H))


def _sc_scatter_body(m_hbm, col_hbm, sums_hbm,
                     colv, mbuf0, mbuf1, sums_sh, sm0, sm1):
    cid = lax.axis_index("c")
    sid = lax.axis_index("s")
    wid = cid * 16 + sid
    base = wid * EP

    pltpu.sync_copy(col_hbm.at[wid], colv)

    @pl.loop(0, C)
    def _z(i):
        for k in range(H // 16):
            mbuf0[i, pl.ds(k * 16, 16)] = jnp.zeros((16,), jnp.float32)

    for j in range(TPN // C):
        pltpu.sync_copy(mbuf0, sums_sh.at[pl.ds(sid * TPN + j * C, C)])
    plsc.subcore_barrier()

    def _load(c, mb, sm):
        pltpu.async_copy(m_hbm.at[pl.ds(base + c * C, C)], mb, sm)

    def _load_wait(c, mb, sm):
        pltpu.make_async_copy(m_hbm.at[pl.ds(base + c * C, C)], mb, sm).wait()

    _load(0, mbuf0, sm0)

    @pl.loop(0, NCH // 2)
    def _pair(k):
        c0 = k * 2
        _load(c0 + 1, mbuf1, sm1)
        _load_wait(c0, mbuf0, sm0)
        pltpu.sync_copy(mbuf0, sums_sh.at[colv.at[c0]], add=True)
        _load(c0 + 2, mbuf0, sm0)
        _load_wait(c0 + 1, mbuf1, sm1)
        pltpu.sync_copy(mbuf1, sums_sh.at[colv.at[c0 + 1]], add=True)

    cl = NCH - 1
    _load_wait(cl, mbuf0, sm0)
    pltpu.sync_copy(mbuf0, sums_sh.at[colv.at[cl]], add=True)

    plsc.subcore_barrier()
    pltpu.sync_copy(sums_sh.at[pl.ds(sid * TPN, TPN)],
                    sums_hbm.at[cid, pl.ds(sid * TPN, TPN)])


def _sc_scatter(m, col_r):
    f = pl.kernel(
        _sc_scatter_body,
        out_type=jax.ShapeDtypeStruct((2, NP, H), jnp.float32),
        mesh=_mesh(),
        scratch_types=[
            pltpu.VMEM((NCH, C), jnp.int32),
            pltpu.VMEM((C, H), jnp.float32),
            pltpu.VMEM((C, H), jnp.float32),
            pltpu.VMEM_SHARED((NP, H), jnp.float32),
            pltpu.SemaphoreType.DMA,
            pltpu.SemaphoreType.DMA,
        ],
        compiler_params=pltpu.CompilerParams(needs_layout_passes=False),
    )
    return f(m, col_r)


def _final_body(h_ref, hn_ref, s_ref, c_ref, wn1a_ref, wn1b_ref, bn1_ref,
                wn2_ref, bn2_ref, wm1_ref, bm1_ref, wm2_ref, bm2_ref,
                g2_ref, beta2_ref, out_ref):
    sums = s_ref[0] + s_ref[1]
    cnt = c_ref[0, 0] + c_ref[0, 1]
    m_aggr = sums / jnp.maximum(cnt, 1.0)[:, None]
    hn = hn_ref[...]
    pre_n = (jnp.dot(hn, wn1a_ref[...], preferred_element_type=jnp.float32)
             + jnp.dot(m_aggr, wn1b_ref[...], preferred_element_type=jnp.float32)
             + bn1_ref[...])
    h_delta = jnp.dot(_silu(pre_n), wn2_ref[...],
                      preferred_element_type=jnp.float32) + bn2_ref[...]
    h1 = h_ref[...] + hn + h_delta
    h2n = _layernorm(h1, g2_ref[...], beta2_ref[...])
    h_mlp = jnp.dot(_silu(jnp.dot(h2n, wm1_ref[...],
                                  preferred_element_type=jnp.float32)
                          + bm1_ref[...]),
                    wm2_ref[...], preferred_element_type=jnp.float32) + bm2_ref[...]
    out_ref[...] = h1 + h_mlp


def _tc_final(h_p, hn, sums2, cnt2, wn1a, wn1b, b_n1, w_n2, b_n2,
              w_m1, b_m1, w_m2, b_m2, g2, beta2):
    BN = 1024
    grid = NP // BN
    cnt3 = cnt2.reshape(1, 2, NP)
    return pl.pallas_call(
        _final_body,
        grid=(grid,),
        in_specs=[
            pl.BlockSpec((BN, D), lambda i: (i, 0)),
            pl.BlockSpec((BN, D), lambda i: (i, 0)),
            pl.BlockSpec((2, BN, H), lambda i: (0, i, 0)),
            pl.BlockSpec((1, 2, BN), lambda i: (0, 0, i)),
            pl.BlockSpec((D, H), lambda i: (0, 0)),
            pl.BlockSpec((H, H), lambda i: (0, 0)),
            pl.BlockSpec((1, H), lambda i: (0, 0)),
            pl.BlockSpec((H, D), lambda i: (0, 0)),
            pl.BlockSpec((1, D), lambda i: (0, 0)),
            pl.BlockSpec((D, H), lambda i: (0, 0)),
            pl.BlockSpec((1, H), lambda i: (0, 0)),
            pl.BlockSpec((H, D), lambda i: (0, 0)),
            pl.BlockSpec((1, D), lambda i: (0, 0)),
            pl.BlockSpec((1, D), lambda i: (0, 0)),
            pl.BlockSpec((1, D), lambda i: (0, 0)),
        ],
        out_specs=pl.BlockSpec((BN, D), lambda i: (i, 0)),
        out_shape=jax.ShapeDtypeStruct((NP, D), jnp.float32),
    )(h_p, hn, sums2, cnt3, wn1a, wn1b, b_n1.reshape(1, H), w_n2,
      b_n2.reshape(1, D), w_m1, b_m1.reshape(1, H), w_m2,
      b_m2.reshape(1, D), g2.reshape(1, D), beta2.reshape(1, D))


def kernel(x, h, edge_index, W_e1, b_e1, W_e2, b_e2, W_n1, b_n1, W_n2, b_n2,
           W_m1, b_m1, W_m2, b_m2, g1, beta1, g2, beta2):
    row = edge_index[0].astype(jnp.int32)
    col = edge_index[1].astype(jnp.int32)
    row_r = row.reshape(NW, NCH, C)
    col_r = col.reshape(NW, NCH, C)

    h_p = jnp.zeros((NP, D), jnp.float32).at[:N].set(h)
    x0, x1, x2 = x[:, 0], x[:, 1], x[:, 2]

    wa = W_e1[:D]
    wb = W_e1[D:2 * D]
    wd = W_e1[2 * D]

    hn, a, b = _tc_prep(h_p, g1, beta1, wa, wb, b_e1)
    abp = lax.bitcast_convert_type(
        jnp.concatenate([a, b], axis=1).reshape(NP, H, 2), jnp.int32)
    wdi = lax.bitcast_convert_type(
        wd.astype(jnp.bfloat16).reshape(H // 2, 2), jnp.int32)
    p = np.arange(H)
    f, j = p // 32, p % 32
    perm = np.where(j < 16, 32 * f + 2 * j, 32 * f + 2 * (j - 16) + 1)
    pre, cnt2 = _sc_gather(abp, x0, x1, x2, row_r, col_r, wdi)
    m = _tc_edge(pre, W_e2[perm], b_e2)
    sums2 = _sc_scatter(m, col_r)
    out_p = _tc_final(h_p, hn, sums2, cnt2, W_n1[:D], W_n1[D:], b_n1,
                      W_n2, b_n2, W_m1, b_m1, W_m2, b_m2, g2, beta2)
    return out_p[:N]


# speedup vs baseline: 1.9923x; 1.0040x over previous
import functools
import jax
import jax.numpy as jnp
import numpy as np
from jax import lax
from jax.experimental import pallas as pl
from jax.experimental.pallas import tpu as pltpu
from jax.experimental.pallas import tpu_sc as plsc

N = 10000
NP = 10240
E = 320000
D = 128
H = 128

NW = 32
EP = E // NW
C = 80
NCH = EP // C
TPN = NP // 16

@functools.cache
def _mesh():
    return plsc.VectorSubcoreMesh(core_axis_name="c", subcore_axis_name="s")


def _silu(v):
    return v * jax.nn.sigmoid(v)


def _layernorm(v, g, b, eps=1e-5):
    mu = jnp.mean(v, axis=-1, keepdims=True)
    var = jnp.mean((v - mu) ** 2, axis=-1, keepdims=True)
    return (v - mu) / jnp.sqrt(var + eps) * g + b


def _prep_body(h_ref, g1_ref, b1_ref, wa_ref, wb_ref, be1_ref,
               hn_ref, a_ref, b_ref):
    hn = _layernorm(h_ref[...], g1_ref[...], b1_ref[...])
    hn_ref[...] = hn
    a_ref[...] = (jnp.dot(hn, wa_ref[...], preferred_element_type=jnp.float32)
                  + be1_ref[...]).astype(jnp.bfloat16)
    b_ref[...] = jnp.dot(hn, wb_ref[...],
                         preferred_element_type=jnp.float32).astype(jnp.bfloat16)


def _tc_prep(h_p, g1, beta1, wa, wb, be1):
    BN = 1024
    grid = NP // BN
    return pl.pallas_call(
        _prep_body,
        grid=(grid,),
        in_specs=[
            pl.BlockSpec((BN, D), lambda i: (i, 0)),
            pl.BlockSpec((1, D), lambda i: (0, 0)),
            pl.BlockSpec((1, D), lambda i: (0, 0)),
            pl.BlockSpec((D, H), lambda i: (0, 0)),
            pl.BlockSpec((D, H), lambda i: (0, 0)),
            pl.BlockSpec((1, H), lambda i: (0, 0)),
        ],
        out_specs=[
            pl.BlockSpec((BN, D), lambda i: (i, 0)),
            pl.BlockSpec((BN, H), lambda i: (i, 0)),
            pl.BlockSpec((BN, H), lambda i: (i, 0)),
        ],
        out_shape=[
            jax.ShapeDtypeStruct((NP, D), jnp.float32),
            jax.ShapeDtypeStruct((NP, H), jnp.bfloat16),
            jax.ShapeDtypeStruct((NP, H), jnp.bfloat16),
        ],
    )(h_p, g1.reshape(1, D), beta1.reshape(1, D), wa, wb, be1.reshape(1, H))


def _rsqrt_sc(r2):
    y = plsc.bitcast(r2, jnp.int32)
    y = jnp.int32(0x5F3759DF) - lax.shift_right_logical(y, 1)
    g = plsc.bitcast(y, jnp.float32)
    g = g * (1.5 - 0.5 * r2 * g * g)
    g = g * (1.5 - 0.5 * r2 * g * g)
    g = g * (1.5 - 0.5 * r2 * g * g)
    return g


def _sc_gather_body(abp_hbm, x0_hbm, x1_hbm, x2_hbm, row_hbm, col_hbm,
                    wd_hbm, pre_hbm, cnt_hbm,
                    rowv, colv, x0v, x1v, x2v, wdv, onesv,
                    abuf0, abuf1, bbuf0, bbuf1, obuf0, obuf1, dv0, dv1, zbuf,
                    cnt_sh, sa0, sa1, sb0, sb1, so0, so1, scnt):
    cid = lax.axis_index("c")
    sid = lax.axis_index("s")
    wid = cid * 16 + sid
    base = wid * EP

    pltpu.sync_copy(row_hbm.at[wid], rowv)
    pltpu.sync_copy(col_hbm.at[wid], colv)
    pltpu.sync_copy(x0_hbm, x0v)
    pltpu.sync_copy(x1_hbm, x1v)
    pltpu.sync_copy(x2_hbm, x2v)
    pltpu.sync_copy(wd_hbm, wdv)

    for i in range(TPN // 16):
        zbuf[pl.ds(i * 16, 16)] = jnp.zeros((16,), jnp.float32)
    for i in range(C // 16):
        onesv[pl.ds(i * 16, 16)] = jnp.ones((16,), jnp.float32)
    pltpu.sync_copy(zbuf, cnt_sh.at[pl.ds(sid * TPN, TPN)])
    plsc.subcore_barrier()

    def _dist(c, dv):
        rv = rowv.at[c]
        cv = colv.at[c]
        for g in range(C // 16):
            r16 = rv[pl.ds(g * 16, 16)]
            c16 = cv[pl.ds(g * 16, 16)]
            dx = plsc.load_gather(x0v, [r16]) - plsc.load_gather(x0v, [c16])
            dy = plsc.load_gather(x1v, [r16]) - plsc.load_gather(x1v, [c16])
            dz = plsc.load_gather(x2v, [r16]) - plsc.load_gather(x2v, [c16])
            r2 = dx * dx + dy * dy + dz * dz
            dv[pl.ds(g * 16, 16)] = r2 * _rsqrt_sc(jnp.maximum(r2, 1e-30))

    def _start(c, ab, bb, sa, sb):
        pltpu.async_copy(abp_hbm.at[rowv.at[c]], ab, sa)
        pltpu.async_copy(abp_hbm.at[colv.at[c]], bb, sb)

    def _wait(c, ab, bb, sa, sb):
        pltpu.make_async_copy(abp_hbm.at[rowv.at[c]], ab, sa).wait()
        pltpu.make_async_copy(abp_hbm.at[colv.at[c]], bb, sb).wait()

    def _compute(ab, bb, ob, dv):
        @pl.loop(0, C, unroll=8)
        def _edge(e):
            dsp = plsc.load_gather(dv, [jnp.full((16,), e, jnp.int32)])
            dspb = plsc.pack(dsp, dsp, format=plsc.PackFormat.INTERLEAVED)
            for f in range(H // 32):
                av = plsc.bitcast(ab[e, pl.ds(f * 16, 16)], jnp.bfloat16)
                bv = plsc.bitcast(bb[e, pl.ds(64 + f * 16, 16)], jnp.bfloat16)
                wv = plsc.bitcast(wdv[pl.ds(f * 16, 16)], jnp.bfloat16)
                res = av + bv + dspb * wv
                lo, hi = plsc.unpack(res, format=plsc.PackFormat.INTERLEAVED)
                ob[e, pl.ds(f * 32, 16)] = lo
                ob[e, pl.ds(f * 32 + 16, 16)] = hi

    def _store(c, ob, so):
        pltpu.async_copy(ob, pre_hbm.at[pl.ds(base + c * C, C)], so)

    def _store_wait(c, ob, so):
        pltpu.make_async_copy(ob, pre_hbm.at[pl.ds(base + c * C, C)], so).wait()

    _start(0, abuf0, bbuf0, sa0, sb0)

    @pl.loop(0, NCH // 2)
    def _pair(k):
        c0 = k * 2
        c1 = c0 + 1
        _start(c1, abuf1, bbuf1, sa1, sb1)
        pltpu.async_copy(onesv, cnt_sh.at[colv.at[c0]], scnt, add=True)
        _dist(c0, dv0)
        _wait(c0, abuf0, bbuf0, sa0, sb0)

        @pl.when(k > 0)
        def _():
            _store_wait(c0, obuf0, so0)

        _compute(abuf0, bbuf0, obuf0, dv0)
        _store(c0, obuf0, so0)

        _start(c0 + 2, abuf0, bbuf0, sa0, sb0)
        pltpu.async_copy(onesv, cnt_sh.at[colv.at[c1]], scnt, add=True)
        _dist(c1, dv1)
        _wait(c1, abuf1, bbuf1, sa1, sb1)

        @pl.when(k > 0)
        def _():
            _store_wait(c1, obuf1, so1)

        _compute(abuf1, bbuf1, obuf1, dv1)
        _store(c1, obuf1, so1)

    cl = NCH - 1
    pltpu.async_copy(onesv, cnt_sh.at[colv.at[cl]], scnt, add=True)
    _dist(cl, dv0)
    _wait(cl, abuf0, bbuf0, sa0, sb0)
    _store_wait(cl, obuf0, so0)
    _compute(abuf0, bbuf0, obuf0, dv0)
    _store(cl, obuf0, so0)
    _store_wait(cl, obuf0, so0)
    _store_wait(cl, obuf1, so1)

    @pl.loop(0, NCH)
    def _drain(c):
        pltpu.make_async_copy(onesv, cnt_sh.at[colv.at[c]], scnt).wait()

    plsc.subcore_barrier()
    pltpu.sync_copy(cnt_sh.at[pl.ds(sid * TPN, TPN)],
                    cnt_hbm.at[cid, pl.ds(sid * TPN, TPN)])


def _sc_gather(abp, x0, x1, x2, row_r, col_r, wdi):
    f = pl.kernel(
        _sc_gather_body,
        out_type=[
            jax.ShapeDtypeStruct((E, H), jnp.float32),
            jax.ShapeDtypeStruct((2, NP), jnp.float32),
        ],
        mesh=_mesh(),
        scratch_types=[
            pltpu.VMEM((NCH, C), jnp.int32),
            pltpu.VMEM((NCH, C), jnp.int32),
            pltpu.VMEM((N,), jnp.float32),
            pltpu.VMEM((N,), jnp.float32),
            pltpu.VMEM((N,), jnp.float32),
            pltpu.VMEM((H // 2,), jnp.int32),
            pltpu.VMEM((C,), jnp.float32),
            pltpu.VMEM((C, H), jnp.int32),
            pltpu.VMEM((C, H), jnp.int32),
            pltpu.VMEM((C, H), jnp.int32),
            pltpu.VMEM((C, H), jnp.int32),
            pltpu.VMEM((C, H), jnp.float32),
            pltpu.VMEM((C, H), jnp.float32),
            pltpu.VMEM((C,), jnp.float32),
            pltpu.VMEM((C,), jnp.float32),
            pltpu.VMEM((TPN,), jnp.float32),
            pltpu.VMEM_SHARED((NP,), jnp.float32),
            pltpu.SemaphoreType.DMA,
            pltpu.SemaphoreType.DMA,
            pltpu.SemaphoreType.DMA,
            pltpu.SemaphoreType.DMA,
            pltpu.SemaphoreType.DMA,
            pltpu.SemaphoreType.DMA,
            pltpu.SemaphoreType.DMA,
        ],
        compiler_params=pltpu.CompilerParams(needs_layout_passes=False,
                                             disable_bounds_checks=True),
    )
    return f(abp, x0, x1, x2, row_r, col_r, wdi)


_BE = 2560


def _edge_body(pre_ref, w2_ref, b2_ref, m_ref):
    t = _silu(pre_ref[...]).astype(jnp.bfloat16)
    m_ref[...] = _silu(jnp.dot(t, w2_ref[...], preferred_element_type=jnp.float32)
                       + b2_ref[...])


def _tc_edge(pre, w2, b2):
    grid = E // _BE
    return pl.pallas_call(
        _edge_body,
        grid=(grid,),
        in_specs=[
            pl.BlockSpec((_BE, H), lambda i: (i, 0)),
            pl.BlockSpec((H, H), lambda i: (0, 0)),
            pl.BlockSpec((1, H), lambda i: (0, 0)),
        ],
        out_specs=pl.BlockSpec((_BE, H), lambda i: (i, 0)),
        out_shape=jax.ShapeDtypeStruct((E, H), jnp.float32),
    )(pre, w2.astype(jnp.bfloat16), b2.reshape(1, H))


def _sc_scatter_body(m_hbm, col_hbm, sums_hbm,
                     colv, mbuf0, mbuf1, sums_sh, sm0, sm1):
    cid = lax.axis_index("c")
    sid = lax.axis_index("s")
    wid = cid * 16 + sid
    base = wid * EP

    pltpu.sync_copy(col_hbm.at[wid], colv)

    @pl.loop(0, C)
    def _z(i):
        for k in range(H // 16):
            mbuf0[i, pl.ds(k * 16, 16)] = jnp.zeros((16,), jnp.float32)

    for j in range(TPN // C):
        pltpu.sync_copy(mbuf0, sums_sh.at[pl.ds(sid * TPN + j * C, C)])
    plsc.subcore_barrier()

    def _load(c, mb, sm):
        pltpu.async_copy(m_hbm.at[pl.ds(base + c * C, C)], mb, sm)

    def _load_wait(c, mb, sm):
        pltpu.make_async_copy(m_hbm.at[pl.ds(base + c * C, C)], mb, sm).wait()

    _load(0, mbuf0, sm0)

    @pl.loop(0, NCH // 2)
    def _pair(k):
        c0 = k * 2
        _load(c0 + 1, mbuf1, sm1)
        _load_wait(c0, mbuf0, sm0)
        pltpu.sync_copy(mbuf0, sums_sh.at[colv.at[c0]], add=True)
        _load(c0 + 2, mbuf0, sm0)
        _load_wait(c0 + 1, mbuf1, sm1)
        pltpu.sync_copy(mbuf1, sums_sh.at[colv.at[c0 + 1]], add=True)

    cl = NCH - 1
    _load_wait(cl, mbuf0, sm0)
    pltpu.sync_copy(mbuf0, sums_sh.at[colv.at[cl]], add=True)

    plsc.subcore_barrier()
    pltpu.sync_copy(sums_sh.at[pl.ds(sid * TPN, TPN)],
                    sums_hbm.at[cid, pl.ds(sid * TPN, TPN)])


def _sc_scatter(m, col_r):
    f = pl.kernel(
        _sc_scatter_body,
        out_type=jax.ShapeDtypeStruct((2, NP, H), jnp.float32),
        mesh=_mesh(),
        scratch_types=[
            pltpu.VMEM((NCH, C), jnp.int32),
            pltpu.VMEM((C, H), jnp.float32),
            pltpu.VMEM((C, H), jnp.float32),
            pltpu.VMEM_SHARED((NP, H), jnp.float32),
            pltpu.SemaphoreType.DMA,
            pltpu.SemaphoreType.DMA,
        ],
        compiler_params=pltpu.CompilerParams(needs_layout_passes=False),
    )
    return f(m, col_r)


def _final_body(h_ref, hn_ref, s_ref, c_ref, wn1a_ref, wn1b_ref, bn1_ref,
                wn2_ref, bn2_ref, wm1_ref, bm1_ref, wm2_ref, bm2_ref,
                g2_ref, beta2_ref, out_ref):
    sums = s_ref[0] + s_ref[1]
    cnt = c_ref[0, 0] + c_ref[0, 1]
    m_aggr = sums / jnp.maximum(cnt, 1.0)[:, None]
    hn = hn_ref[...]
    pre_n = (jnp.dot(hn, wn1a_ref[...], preferred_element_type=jnp.float32)
             + jnp.dot(m_aggr, wn1b_ref[...], preferred_element_type=jnp.float32)
             + bn1_ref[...])
    h_delta = jnp.dot(_silu(pre_n), wn2_ref[...],
                      preferred_element_type=jnp.float32) + bn2_ref[...]
    h1 = h_ref[...] + hn + h_delta
    h2n = _layernorm(h1, g2_ref[...], beta2_ref[...])
    h_mlp = jnp.dot(_silu(jnp.dot(h2n, wm1_ref[...],
                                  preferred_element_type=jnp.float32)
                          + bm1_ref[...]),
                    wm2_ref[...], preferred_element_type=jnp.float32) + bm2_ref[...]
    out_ref[...] = h1 + h_mlp


def _tc_final(h_p, hn, sums2, cnt2, wn1a, wn1b, b_n1, w_n2, b_n2,
              w_m1, b_m1, w_m2, b_m2, g2, beta2):
    BN = 1024
    grid = NP // BN
    cnt3 = cnt2.reshape(1, 2, NP)
    return pl.pallas_call(
        _final_body,
        grid=(grid,),
        in_specs=[
            pl.BlockSpec((BN, D), lambda i: (i, 0)),
            pl.BlockSpec((BN, D), lambda i: (i, 0)),
            pl.BlockSpec((2, BN, H), lambda i: (0, i, 0)),
            pl.BlockSpec((1, 2, BN), lambda i: (0, 0, i)),
            pl.BlockSpec((D, H), lambda i: (0, 0)),
            pl.BlockSpec((H, H), lambda i: (0, 0)),
            pl.BlockSpec((1, H), lambda i: (0, 0)),
            pl.BlockSpec((H, D), lambda i: (0, 0)),
            pl.BlockSpec((1, D), lambda i: (0, 0)),
            pl.BlockSpec((D, H), lambda i: (0, 0)),
            pl.BlockSpec((1, H), lambda i: (0, 0)),
            pl.BlockSpec((H, D), lambda i: (0, 0)),
            pl.BlockSpec((1, D), lambda i: (0, 0)),
            pl.BlockSpec((1, D), lambda i: (0, 0)),
            pl.BlockSpec((1, D), lambda i: (0, 0)),
        ],
        out_specs=pl.BlockSpec((BN, D), lambda i: (i, 0)),
        out_shape=jax.ShapeDtypeStruct((NP, D), jnp.float32),
    )(h_p, hn, sums2, cnt3, wn1a, wn1b, b_n1.reshape(1, H), w_n2,
      b_n2.reshape(1, D), w_m1, b_m1.reshape(1, H), w_m2,
      b_m2.reshape(1, D), g2.reshape(1, D), beta2.reshape(1, D))


def kernel(x, h, edge_index, W_e1, b_e1, W_e2, b_e2, W_n1, b_n1, W_n2, b_n2,
           W_m1, b_m1, W_m2, b_m2, g1, beta1, g2, beta2):
    row = edge_index[0].astype(jnp.int32)
    col = edge_index[1].astype(jnp.int32)
    row_r = row.reshape(NW, NCH, C)
    col_r = col.reshape(NW, NCH, C)

    h_p = jnp.zeros((NP, D), jnp.float32).at[:N].set(h)
    x0, x1, x2 = x[:, 0], x[:, 1], x[:, 2]

    wa = W_e1[:D]
    wb = W_e1[D:2 * D]
    wd = W_e1[2 * D]

    hn, a, b = _tc_prep(h_p, g1, beta1, wa, wb, b_e1)
    abp = lax.bitcast_convert_type(
        jnp.concatenate([a, b], axis=1).reshape(NP, H, 2), jnp.int32)
    wdi = lax.bitcast_convert_type(
        wd.astype(jnp.bfloat16).reshape(H // 2, 2), jnp.int32)
    p = np.arange(H)
    f, j = p // 32, p % 32
    perm = np.where(j < 16, 32 * f + 2 * j, 32 * f + 2 * (j - 16) + 1)
    pre, cnt2 = _sc_gather(abp, x0, x1, x2, row_r, col_r, wdi)
    m = _tc_edge(pre, W_e2[perm], b_e2)
    sums2 = _sc_scatter(m, col_r)
    out_p = _tc_final(h_p, hn, sums2, cnt2, W_n1[:D], W_n1[D:], b_n1,
                      W_n2, b_n2, W_m1, b_m1, W_m2, b_m2, g2, beta2)
    return out_p[:N]
